# Initial kernel scaffold; baseline (speedup 1.0000x reference)
import jax
import jax.numpy as jnp
from jax.experimental import pallas as pl


def kernel(x, edge_index, edge_attr, batch, W1, b1, W2, b2, Wro, bro, Wfc1, bfc1, gamma, beta, Wfc2, bfc2):
    raise NotImplementedError("write your pallas kernel here")


# speedup vs baseline: 2.7730x; 2.7730x over previous
import functools
import jax
import jax.numpy as jnp
from jax import lax
from jax.experimental import pallas as pl
from jax.experimental.pallas import tpu as pltpu

N = 50000
E = 800000
NUMROI = 100
C1 = 128
C2 = 64
B = 500
BM = 1000


def _mish(x):
    return x * jnp.tanh(jax.nn.softplus(x))


def _tca_body(x_ref, w1_ref, deg_ref, hs_ref, dis_ref):
    dis = lax.rsqrt(deg_ref[...] + 1.0)
    h = jnp.dot(x_ref[...], w1_ref[...], preferred_element_type=jnp.float32)
    hs_ref[...] = h * dis
    dis_ref[...] = dis


def _tc_a(x, W1, deg):
    grid = N // BM
    return pl.pallas_call(
        _tca_body,
        grid=(grid,),
        in_specs=[
            pl.BlockSpec((BM, NUMROI), lambda i: (i, 0)),
            pl.BlockSpec((NUMROI, C1), lambda i: (0, 0)),
            pl.BlockSpec((BM, 1), lambda i: (i, 0)),
        ],
        out_specs=[
            pl.BlockSpec((BM, C1), lambda i: (i, 0)),
            pl.BlockSpec((BM, 1), lambda i: (i, 0)),
        ],
        out_shape=[
            jax.ShapeDtypeStruct((N, C1), jnp.float32),
            jax.ShapeDtypeStruct((N, 1), jnp.float32),
        ],
    )(x, W1, deg)


def _tcb_body(acc_ref, hs_ref, dis_ref, w2_ref, b1_ref, hs2_ref):
    dis = dis_ref[...]
    z = dis * (acc_ref[...] + hs_ref[...]) + b1_ref[...]
    g = _mish(z)
    hs2_ref[...] = jnp.dot(g, w2_ref[...], preferred_element_type=jnp.float32) * dis


def _tc_b(acc1, hs1, dis, W2, b1):
    grid = N // BM
    return pl.pallas_call(
        _tcb_body,
        grid=(grid,),
        in_specs=[
            pl.BlockSpec((BM, C1), lambda i: (i, 0)),
            pl.BlockSpec((BM, C1), lambda i: (i, 0)),
            pl.BlockSpec((BM, 1), lambda i: (i, 0)),
            pl.BlockSpec((C1, C2), lambda i: (0, 0)),
            pl.BlockSpec((1, C1), lambda i: (0, 0)),
        ],
        out_specs=pl.BlockSpec((BM, C2), lambda i: (i, 0)),
        out_shape=jax.ShapeDtypeStruct((N, C2), jnp.float32),
    )(acc1, hs1, dis, W2, b1)


def _tcc_body(acc_ref, hs_ref, dis_ref, wro_ref, b2_ref, bro_ref, r_ref):
    z = dis_ref[...] * (acc_ref[...] + hs_ref[...]) + b2_ref[...]
    g = _mish(z)
    r_ref[...] = _mish(
        jnp.dot(g, wro_ref[...], preferred_element_type=jnp.float32) + bro_ref[...]
    )


def _tc_c(acc2, hs2, dis, Wro, b2, bro):
    grid = N // BM
    return pl.pallas_call(
        _tcc_body,
        grid=(grid,),
        in_specs=[
            pl.BlockSpec((BM, C2), lambda i: (i, 0)),
            pl.BlockSpec((BM, C2), lambda i: (i, 0)),
            pl.BlockSpec((BM, 1), lambda i: (i, 0)),
            pl.BlockSpec((C2, 8), lambda i: (0, 0)),
            pl.BlockSpec((1, C2), lambda i: (0, 0)),
            pl.BlockSpec((1, 8), lambda i: (0, 0)),
        ],
        out_specs=pl.BlockSpec((BM, 8), lambda i: (i, 0)),
        out_shape=jax.ShapeDtypeStruct((N, 8), jnp.float32),
    )(acc2, hs2, dis, Wro, b2, bro)


def _tcd_body(f_ref, wfc1_ref, bfc1_ref, g_ref, be_ref, wfc2_ref, bfc2_ref, o_ref):
    z = (
        jnp.dot(f_ref[...], wfc1_ref[...], preferred_element_type=jnp.float32)
        + bfc1_ref[...]
    )
    mu = jnp.mean(z, axis=0, keepdims=True)
    var = jnp.mean((z - mu) ** 2, axis=0, keepdims=True)
    zn = (z - mu) / jnp.sqrt(var + 1e-5) * g_ref[...] + be_ref[...]
    o_ref[...] = (
        jnp.dot(_mish(zn), wfc2_ref[...], preferred_element_type=jnp.float32)
        + bfc2_ref[...]
    )


def _tc_d(feats, Wfc1, bfc1, gamma, beta, Wfc2, bfc2):
    F = NUMROI * 8
    return pl.pallas_call(
        _tcd_body,
        in_specs=[
            pl.BlockSpec((B, F), lambda: (0, 0)),
            pl.BlockSpec((F, NUMROI), lambda: (0, 0)),
            pl.BlockSpec((1, NUMROI), lambda: (0, 0)),
            pl.BlockSpec((1, NUMROI), lambda: (0, 0)),
            pl.BlockSpec((1, NUMROI), lambda: (0, 0)),
            pl.BlockSpec((NUMROI, 2), lambda: (0, 0)),
            pl.BlockSpec((1, 2), lambda: (0, 0)),
        ],
        out_specs=pl.BlockSpec((B, 2), lambda: (0, 0)),
        out_shape=jax.ShapeDtypeStruct((B, 2), jnp.float32),
    )(feats, Wfc1, bfc1, gamma, beta, Wfc2, bfc2)


def _deg_jnp(col, w):
    return jnp.zeros((N,), jnp.float32).at[col].add(w)


def _edge_jnp(hs, row, col, w):
    return jnp.zeros_like(hs).at[col].add(w[:, None] * hs[row])


def kernel(x, edge_index, edge_attr, batch, W1, b1, W2, b2, Wro, bro,
           Wfc1, bfc1, gamma, beta, Wfc2, bfc2):
    row = edge_index[0].astype(jnp.int32)
    col = edge_index[1].astype(jnp.int32)
    w = edge_attr

    deg = _deg_jnp(col, w)
    hs1, dis = _tc_a(x, W1, deg.reshape(-1, 1))
    acc1 = _edge_jnp(hs1, row, col, w)
    hs2 = _tc_b(acc1, hs1, dis, W2, b1.reshape(1, -1))
    acc2 = _edge_jnp(hs2, row, col, w)
    r = _tc_c(acc2, hs2, dis, Wro, b2.reshape(1, -1), bro.reshape(1, -1))
    feats = r.reshape(B, NUMROI * 8)
    return _tc_d(feats, Wfc1, bfc1.reshape(1, -1), gamma.reshape(1, -1),
                 beta.reshape(1, -1), Wfc2, bfc2.reshape(1, -1))


# speedup vs baseline: 2.9008x; 1.0461x over previous
import functools
import jax
import jax.numpy as jnp
from jax import lax
from jax.experimental import pallas as pl
from jax.experimental.pallas import tpu as pltpu
from jax.experimental.pallas import tpu_sc as plsc

N = 50000
E = 800000
NUMROI = 100
C1 = 128
C2 = 64
B = 500
BM = 1000

NPAD = 50176
EPS = E // 16
SUB = 2000


_NH = NPAD // 2
_RPT_D = _NH // 16


def _sc_deg_body(col_hbm, w_hbm, deg_hbm, colb, wb, zb, deg_sh):
    core = lax.axis_index("c")
    sub = lax.axis_index("s")
    lo = core * _NH

    def _z(i, _):
        zb[pl.ds(i * 16, 16)] = jnp.zeros((16,), jnp.float32)
        return 0

    lax.fori_loop(0, _RPT_D // 16, _z, 0)
    pltpu.sync_copy(zb, deg_sh.at[pl.ds(sub * _RPT_D, _RPT_D)])
    plsc.subcore_barrier()

    for c in range(EPS // SUB):
        base = sub * EPS + c * SUB
        pltpu.sync_copy(col_hbm.at[pl.ds(base, SUB)], colb)
        pltpu.sync_copy(w_hbm.at[pl.ds(base, SUB)], wb)

        def _mask(i, _):
            sl = pl.ds(i * 16, 16)
            cv = colb[sl]
            wv = wb[sl]
            m = (cv >= lo) & (cv < lo + _NH)
            colb[sl] = jnp.where(m, cv - lo, 0)
            wb[sl] = jnp.where(m, wv, 0.0)
            return 0

        lax.fori_loop(0, SUB // 16, _mask, 0)
        pltpu.sync_copy(wb, deg_sh.at[colb], add=True)

    plsc.subcore_barrier()
    pltpu.sync_copy(deg_sh.at[pl.ds(sub * _RPT_D, _RPT_D)], zb)
    pltpu.sync_copy(zb, deg_hbm.at[pl.ds(lo + sub * _RPT_D, _RPT_D)])


_sc_deg = functools.partial(
    pl.kernel,
    out_type=jax.ShapeDtypeStruct((NPAD,), jnp.float32),
    mesh=plsc.VectorSubcoreMesh(core_axis_name="c", subcore_axis_name="s"),
    scratch_types=[
        pltpu.VMEM((SUB,), jnp.int32),
        pltpu.VMEM((SUB,), jnp.float32),
        pltpu.VMEM((_RPT_D,), jnp.float32),
        pltpu.VMEM_SHARED((_NH,), jnp.float32),
    ],
)(_sc_deg_body)


QW = 32
EPAD = 819200
SUB2 = 2048
_EPT = EPAD // 16
_G = 128
_ZB = 112
_RPT = NPAD // 16


def _take16(v, idx):
    dn = lax.GatherDimensionNumbers(
        offset_dims=(), collapsed_slice_dims=(0,), start_index_map=(0,))
    return lax.gather(v, idx[:, None], dn, (1,),
                      mode=lax.GatherScatterMode.PROMISE_IN_BOUNDS)


def _edge_body(NQ, *refs):
    hs_q = refs[:NQ]
    row_hbm, col_hbm, w_hbm = refs[NQ:NQ + 3]
    acc_q = refs[NQ + 3:2 * NQ + 3]
    rowb, colb, wb, colbuf, zbuf, flshb, rows_v, acc_sh, sem = refs[2 * NQ + 3:]
    core = lax.axis_index("c")
    sub = lax.axis_index("s")

    for zi in range(_ZB):
        for cb in range(QW // 16):
            zbuf[zi, pl.ds(cb * 16, 16)] = jnp.zeros((16,), jnp.float32)

    for p in range(NQ // 2):
        for k in range(_RPT // _ZB):
            pltpu.sync_copy(zbuf, acc_sh.at[pl.ds(sub * _RPT + k * _ZB, _ZB), :])
        plsc.subcore_barrier()

        def _chunk(c):
            base = sub * _EPT + c * SUB2
            pltpu.sync_copy(row_hbm.at[pl.ds(base, SUB2)], rowb)
            pltpu.sync_copy(col_hbm.at[pl.ds(base, SUB2)], colb)
            pltpu.sync_copy(w_hbm.at[pl.ds(base, SUB2)], wb)

            def _group(g, _):
                gb = g * _G
                for k in range(_G // 16):
                    colbuf[pl.ds(k * 16, 16)] = colb[pl.ds(gb + k * 16, 16)]
                for q in (2 * p, 2 * p + 1):
                    @pl.when(core == (q % 2))
                    def _(q=q):
                        pltpu.async_copy(
                            hs_q[q].at[rowb.at[pl.ds(gb, _G)]], rows_v, sem
                        ).wait()
                for jv in range(_G // 16):
                    wv = wb[pl.ds(gb + jv * 16, 16)]
                    for l in range(16):
                        ws = _take16(wv, jnp.full((16,), l, jnp.int32))
                        j = jv * 16 + l
                        for cb in range(QW // 16):
                            sl = pl.ds(cb * 16, 16)
                            rows_v[j, sl] = rows_v[j, sl] * ws
                pltpu.sync_copy(rows_v, acc_sh.at[colbuf], add=True)
                return 0

            lax.fori_loop(0, SUB2 // _G, _group, 0)

        for c in range(_EPT // SUB2):
            _chunk(c)
        plsc.subcore_barrier()
        for q in (2 * p, 2 * p + 1):
            @pl.when(core == (q % 2))
            def _(q=q):
                for k in range(_RPT // _ZB):
                    r0 = sub * _RPT + k * _ZB
                    pltpu.sync_copy(acc_sh.at[pl.ds(r0, _ZB), :], flshb)
                    pltpu.sync_copy(flshb, acc_q[q].at[pl.ds(r0, _ZB), :])
        if p + 1 < NQ // 2:
            plsc.subcore_barrier()


def _sc_edge(NQ):
    return functools.partial(
        pl.kernel,
        out_type=[jax.ShapeDtypeStruct((NPAD, QW), jnp.float32)] * NQ,
        mesh=plsc.VectorSubcoreMesh(core_axis_name="c", subcore_axis_name="s"),
        compiler_params=pltpu.CompilerParams(use_tc_tiling_on_sc=False),
        scratch_types=[
            pltpu.VMEM((SUB2,), jnp.int32),
            pltpu.VMEM((SUB2,), jnp.int32),
            pltpu.VMEM((SUB2,), jnp.float32),
            pltpu.VMEM((_G,), jnp.int32),
            pltpu.VMEM((_ZB, QW), jnp.float32),
            pltpu.VMEM((_ZB, QW), jnp.float32),
            pltpu.VMEM((_G, QW), jnp.float32),
            pltpu.VMEM_SHARED((NPAD, QW), jnp.float32),
            pltpu.SemaphoreType.DMA,
        ],
    )(functools.partial(_edge_body, NQ))


_sc_edge1 = _sc_edge(4)
_sc_edge2 = _sc_edge(2)


def _mish(x):
    return x * jnp.tanh(jax.nn.softplus(x))


def _tca_body(x_ref, w1_ref, deg_ref, h0, h1, h2, h3, dis_ref):
    dis = lax.rsqrt(deg_ref[...] + 1.0)
    h = jnp.dot(x_ref[...], w1_ref[...], preferred_element_type=jnp.float32)
    for q, href in enumerate((h0, h1, h2, h3)):
        href[...] = h[:, q * QW:(q + 1) * QW] * dis
    dis_ref[...] = dis


def _tc_a(x, W1, deg):
    grid = N // BM
    return pl.pallas_call(
        _tca_body,
        grid=(grid,),
        in_specs=[
            pl.BlockSpec((BM, NUMROI), lambda i: (i, 0)),
            pl.BlockSpec((NUMROI, C1), lambda i: (0, 0)),
            pl.BlockSpec((BM, 1), lambda i: (i, 0)),
        ],
        out_specs=[pl.BlockSpec((BM, QW), lambda i: (i, 0))] * 4
        + [pl.BlockSpec((BM, 1), lambda i: (i, 0))],
        out_shape=[jax.ShapeDtypeStruct((N, QW), jnp.float32)] * 4
        + [jax.ShapeDtypeStruct((N, 1), jnp.float32)],
    )(x, W1, deg)


def _tcb_body(a0, a1, a2, a3, h0, h1, h2, h3, dis_ref, w2_ref, b1_ref, o0, o1):
    dis = dis_ref[...]
    z = jnp.concatenate(
        [dis * (a[...] + h[...])
         for a, h in ((a0, h0), (a1, h1), (a2, h2), (a3, h3))],
        axis=1,
    ) + b1_ref[...]
    g = _mish(z)
    hh = jnp.dot(g, w2_ref[...], preferred_element_type=jnp.float32) * dis
    o0[...] = hh[:, :QW]
    o1[...] = hh[:, QW:]


def _tc_b(acc1q, hs1q, dis, W2, b1):
    grid = N // BM
    return pl.pallas_call(
        _tcb_body,
        grid=(grid,),
        in_specs=[pl.BlockSpec((BM, QW), lambda i: (i, 0))] * 8
        + [
            pl.BlockSpec((BM, 1), lambda i: (i, 0)),
            pl.BlockSpec((C1, C2), lambda i: (0, 0)),
            pl.BlockSpec((1, C1), lambda i: (0, 0)),
        ],
        out_specs=[pl.BlockSpec((BM, QW), lambda i: (i, 0))] * 2,
        out_shape=[jax.ShapeDtypeStruct((N, QW), jnp.float32)] * 2,
    )(*acc1q, *hs1q, dis, W2, b1)


def _tcc_body(a0, a1, h0, h1, dis_ref, wro_ref, b2_ref, bro_ref, r_ref):
    dis = dis_ref[...]
    z = jnp.concatenate(
        [dis * (a[...] + h[...]) for a, h in ((a0, h0), (a1, h1))], axis=1
    ) + b2_ref[...]
    g = _mish(z)
    r_ref[...] = _mish(
        jnp.dot(g, wro_ref[...], preferred_element_type=jnp.float32) + bro_ref[...]
    )


def _tc_c(acc2q, hs2q, dis, Wro, b2, bro):
    grid = N // BM
    return pl.pallas_call(
        _tcc_body,
        grid=(grid,),
        in_specs=[pl.BlockSpec((BM, QW), lambda i: (i, 0))] * 4
        + [
            pl.BlockSpec((BM, 1), lambda i: (i, 0)),
            pl.BlockSpec((C2, 8), lambda i: (0, 0)),
            pl.BlockSpec((1, C2), lambda i: (0, 0)),
            pl.BlockSpec((1, 8), lambda i: (0, 0)),
        ],
        out_specs=pl.BlockSpec((BM, 8), lambda i: (i, 0)),
        out_shape=jax.ShapeDtypeStruct((N, 8), jnp.float32),
    )(*acc2q, *hs2q, dis, Wro, b2, bro)


def _tcd_body(f_ref, wfc1_ref, bfc1_ref, g_ref, be_ref, wfc2_ref, bfc2_ref, o_ref):
    z = (
        jnp.dot(f_ref[...], wfc1_ref[...], preferred_element_type=jnp.float32)
        + bfc1_ref[...]
    )
    mu = jnp.mean(z, axis=0, keepdims=True)
    var = jnp.mean((z - mu) ** 2, axis=0, keepdims=True)
    zn = (z - mu) / jnp.sqrt(var + 1e-5) * g_ref[...] + be_ref[...]
    o_ref[...] = (
        jnp.dot(_mish(zn), wfc2_ref[...], preferred_element_type=jnp.float32)
        + bfc2_ref[...]
    )


def _tc_d(feats, Wfc1, bfc1, gamma, beta, Wfc2, bfc2):
    F = NUMROI * 8
    return pl.pallas_call(
        _tcd_body,
        in_specs=[
            pl.BlockSpec((B, F), lambda: (0, 0)),
            pl.BlockSpec((F, NUMROI), lambda: (0, 0)),
            pl.BlockSpec((1, NUMROI), lambda: (0, 0)),
            pl.BlockSpec((1, NUMROI), lambda: (0, 0)),
            pl.BlockSpec((1, NUMROI), lambda: (0, 0)),
            pl.BlockSpec((NUMROI, 2), lambda: (0, 0)),
            pl.BlockSpec((1, 2), lambda: (0, 0)),
        ],
        out_specs=pl.BlockSpec((B, 2), lambda: (0, 0)),
        out_shape=jax.ShapeDtypeStruct((B, 2), jnp.float32),
    )(feats, Wfc1, bfc1, gamma, beta, Wfc2, bfc2)


def kernel(x, edge_index, edge_attr, batch, W1, b1, W2, b2, Wro, bro,
           Wfc1, bfc1, gamma, beta, Wfc2, bfc2):
    row = edge_index[0].astype(jnp.int32)
    col = edge_index[1].astype(jnp.int32)
    w = edge_attr
    rowp = jnp.pad(row, (0, EPAD - E))
    colp = jnp.pad(col, (0, EPAD - E))
    wp = jnp.pad(w, (0, EPAD - E))

    deg = _sc_deg(col, w)
    *hs1q, dis = _tc_a(x, W1, deg[:N].reshape(-1, 1))
    hs1 = jnp.concatenate(hs1q, axis=1)
    acc1 = jnp.zeros_like(hs1).at[col].add(w[:, None] * hs1[row])
    acc1q = [acc1[:, q * QW:(q + 1) * QW] for q in range(4)]
    hs2q = _tc_b(acc1q, hs1q, dis, W2, b1.reshape(1, -1))
    hs2 = jnp.concatenate(hs2q, axis=1)
    acc2 = jnp.zeros_like(hs2).at[col].add(w[:, None] * hs2[row])
    acc2q = [acc2[:, q * QW:(q + 1) * QW] for q in range(2)]
    r = _tc_c(acc2q, hs2q, dis, Wro, b2.reshape(1, -1), bro.reshape(1, -1))
    feats = r.reshape(B, NUMROI * 8)
    return _tc_d(feats, Wfc1, bfc1.reshape(1, -1), gamma.reshape(1, -1),
                 beta.reshape(1, -1), Wfc2, bfc2.reshape(1, -1))


# speedup vs baseline: 7.8167x; 2.6947x over previous
import functools
import jax
import jax.numpy as jnp
from jax import lax
from jax.experimental import pallas as pl
from jax.experimental.pallas import tpu as pltpu
from jax.experimental.pallas import tpu_sc as plsc

N = 50000
E = 800000
NUMROI = 100
C1 = 128
C2 = 64
B = 500
BM = 1000

NPAD = 50176
EPS = E // 16
SUB = 2000


_NH = NPAD // 2
_RPT_D = _NH // 16


def _sc_deg_body(col_hbm, w_hbm, deg_hbm, colb, wb, zb, deg_sh):
    core = lax.axis_index("c")
    sub = lax.axis_index("s")
    lo = core * _NH

    def _z(i, _):
        zb[pl.ds(i * 16, 16)] = jnp.zeros((16,), jnp.float32)
        return 0

    lax.fori_loop(0, _RPT_D // 16, _z, 0)
    pltpu.sync_copy(zb, deg_sh.at[pl.ds(sub * _RPT_D, _RPT_D)])
    plsc.subcore_barrier()

    for c in range(EPS // SUB):
        base = sub * EPS + c * SUB
        pltpu.sync_copy(col_hbm.at[pl.ds(base, SUB)], colb)
        pltpu.sync_copy(w_hbm.at[pl.ds(base, SUB)], wb)

        def _mask(i, _):
            sl = pl.ds(i * 16, 16)
            cv = colb[sl]
            wv = wb[sl]
            m = (cv >= lo) & (cv < lo + _NH)
            colb[sl] = jnp.where(m, cv - lo, 0)
            wb[sl] = jnp.where(m, wv, 0.0)
            return 0

        lax.fori_loop(0, SUB // 16, _mask, 0)
        pltpu.sync_copy(wb, deg_sh.at[colb], add=True)

    plsc.subcore_barrier()
    pltpu.sync_copy(deg_sh.at[pl.ds(sub * _RPT_D, _RPT_D)], zb)
    pltpu.sync_copy(zb, deg_hbm.at[pl.ds(lo + sub * _RPT_D, _RPT_D)])


_sc_deg = functools.partial(
    pl.kernel,
    out_type=jax.ShapeDtypeStruct((NPAD,), jnp.float32),
    mesh=plsc.VectorSubcoreMesh(core_axis_name="c", subcore_axis_name="s"),
    scratch_types=[
        pltpu.VMEM((SUB,), jnp.int32),
        pltpu.VMEM((SUB,), jnp.float32),
        pltpu.VMEM((_RPT_D,), jnp.float32),
        pltpu.VMEM_SHARED((_NH,), jnp.float32),
    ],
)(_sc_deg_body)


QW = 32
EPAD = 819200
SUB2 = 2048
_EPT = EPAD // 16
_G = 128
_ZB = 112
_RPT = NPAD // 16


def _take16(v, idx):
    dn = lax.GatherDimensionNumbers(
        offset_dims=(), collapsed_slice_dims=(0,), start_index_map=(0,))
    return lax.gather(v, idx[:, None], dn, (1,),
                      mode=lax.GatherScatterMode.PROMISE_IN_BOUNDS)


def _edge_body(NQ, *refs):
    hs_q = refs[:NQ]
    row_hbm, col_hbm, w_hbm = refs[NQ:NQ + 3]
    acc_out = refs[NQ + 3]
    rowb, colb, wb, colbuf, zbuf, flshb, rows_v, acc_sh, sem = refs[NQ + 4:]
    core = lax.axis_index("c")
    sub = lax.axis_index("s")

    for zi in range(_ZB):
        for cb in range(QW // 16):
            zbuf[zi, pl.ds(cb * 16, 16)] = jnp.zeros((16,), jnp.float32)

    for p in range(NQ // 2):
        for k in range(_RPT // _ZB):
            pltpu.sync_copy(zbuf, acc_sh.at[pl.ds(sub * _RPT + k * _ZB, _ZB), :])
        plsc.subcore_barrier()

        def _chunk(c, _):
            base = sub * _EPT + c * SUB2
            pltpu.sync_copy(row_hbm.at[pl.ds(base, SUB2)], rowb)
            pltpu.sync_copy(col_hbm.at[pl.ds(base, SUB2)], colb)
            pltpu.sync_copy(w_hbm.at[pl.ds(base, SUB2)], wb)

            def _group(g, _):
                gb = g * _G
                for k in range(_G // 16):
                    colbuf[pl.ds(k * 16, 16)] = colb[pl.ds(gb + k * 16, 16)]
                for q in (2 * p, 2 * p + 1):
                    @pl.when(core == (q % 2))
                    def _(q=q):
                        pltpu.async_copy(
                            hs_q[q].at[rowb.at[pl.ds(gb, _G)]], rows_v, sem
                        ).wait()

                def _scale(jv, _):
                    wv = wb[pl.ds(gb + jv * 16, 16)]
                    for l in range(16):
                        ws = _take16(wv, jnp.full((16,), l, jnp.int32))
                        j = jv * 16 + l
                        for cb in range(QW // 16):
                            sl = pl.ds(cb * 16, 16)
                            rows_v[j, sl] = rows_v[j, sl] * ws
                    return 0

                lax.fori_loop(0, _G // 16, _scale, 0)
                pltpu.sync_copy(rows_v, acc_sh.at[colbuf], add=True)
                return 0

            lax.fori_loop(0, SUB2 // _G, _group, 0)
            return 0

        lax.fori_loop(0, _EPT // SUB2, _chunk, 0)
        plsc.subcore_barrier()
        for q in (2 * p, 2 * p + 1):
            @pl.when(core == (q % 2))
            def _(q=q):
                for k in range(_RPT // _ZB):
                    r0 = sub * _RPT + k * _ZB
                    pltpu.sync_copy(acc_sh.at[pl.ds(r0, _ZB), :], flshb)
                    pltpu.sync_copy(
                        flshb,
                        acc_out.at[pl.ds(r0, _ZB), pl.ds(q * QW, QW)])
        if p + 1 < NQ // 2:
            plsc.subcore_barrier()


def _sc_edge(NQ):
    return functools.partial(
        pl.kernel,
        out_type=jax.ShapeDtypeStruct((NPAD, NQ * QW), jnp.float32),
        mesh=plsc.VectorSubcoreMesh(core_axis_name="c", subcore_axis_name="s"),
        compiler_params=pltpu.CompilerParams(use_tc_tiling_on_sc=False),
        scratch_types=[
            pltpu.VMEM((SUB2,), jnp.int32),
            pltpu.VMEM((SUB2,), jnp.int32),
            pltpu.VMEM((SUB2,), jnp.float32),
            pltpu.VMEM((_G,), jnp.int32),
            pltpu.VMEM((_ZB, QW), jnp.float32),
            pltpu.VMEM((_ZB, QW), jnp.float32),
            pltpu.VMEM((_G, QW), jnp.float32),
            pltpu.VMEM_SHARED((NPAD, QW), jnp.float32),
            pltpu.SemaphoreType.DMA,
        ],
    )(functools.partial(_edge_body, NQ))


_sc_edge1 = _sc_edge(4)
_sc_edge2 = _sc_edge(2)


def _mish(x):
    return x * jnp.tanh(jax.nn.softplus(x))


def _tca_body(x_ref, w1_ref, deg_ref, h0, h1, h2, h3, dis_ref):
    dis = lax.rsqrt(deg_ref[...] + 1.0)
    h = jnp.dot(x_ref[...], w1_ref[...], preferred_element_type=jnp.float32)
    for q, href in enumerate((h0, h1, h2, h3)):
        href[...] = h[:, q * QW:(q + 1) * QW] * dis
    dis_ref[...] = dis


def _tc_a(x, W1, deg):
    grid = N // BM
    return pl.pallas_call(
        _tca_body,
        grid=(grid,),
        in_specs=[
            pl.BlockSpec((BM, NUMROI), lambda i: (i, 0)),
            pl.BlockSpec((NUMROI, C1), lambda i: (0, 0)),
            pl.BlockSpec((BM, 1), lambda i: (i, 0)),
        ],
        out_specs=[pl.BlockSpec((BM, QW), lambda i: (i, 0))] * 4
        + [pl.BlockSpec((BM, 1), lambda i: (i, 0))],
        out_shape=[jax.ShapeDtypeStruct((N, QW), jnp.float32)] * 4
        + [jax.ShapeDtypeStruct((N, 1), jnp.float32)],
    )(x, W1, deg)


def _tcb_body(a_ref, h0, h1, h2, h3, dis_ref, w2_ref, b1_ref, o0, o1):
    dis = dis_ref[...]
    hs = jnp.concatenate([h[...] for h in (h0, h1, h2, h3)], axis=1)
    z = dis * (a_ref[...] + hs) + b1_ref[...]
    g = _mish(z)
    hh = jnp.dot(g, w2_ref[...], preferred_element_type=jnp.float32) * dis
    o0[...] = hh[:, :QW]
    o1[...] = hh[:, QW:]


def _tc_b(acc1, hs1q, dis, W2, b1):
    grid = N // BM
    return pl.pallas_call(
        _tcb_body,
        grid=(grid,),
        in_specs=[pl.BlockSpec((BM, C1), lambda i: (i, 0))]
        + [pl.BlockSpec((BM, QW), lambda i: (i, 0))] * 4
        + [
            pl.BlockSpec((BM, 1), lambda i: (i, 0)),
            pl.BlockSpec((C1, C2), lambda i: (0, 0)),
            pl.BlockSpec((1, C1), lambda i: (0, 0)),
        ],
        out_specs=[pl.BlockSpec((BM, QW), lambda i: (i, 0))] * 2,
        out_shape=[jax.ShapeDtypeStruct((N, QW), jnp.float32)] * 2,
    )(acc1, *hs1q, dis, W2, b1)


def _tcc_body(a_ref, h0, h1, dis_ref, wro_ref, b2_ref, bro_ref, r_ref):
    dis = dis_ref[...]
    hs = jnp.concatenate([h[...] for h in (h0, h1)], axis=1)
    z = dis * (a_ref[...] + hs) + b2_ref[...]
    g = _mish(z)
    r_ref[...] = _mish(
        jnp.dot(g, wro_ref[...], preferred_element_type=jnp.float32) + bro_ref[...]
    )


def _tc_c(acc2, hs2q, dis, Wro, b2, bro):
    grid = N // BM
    return pl.pallas_call(
        _tcc_body,
        grid=(grid,),
        in_specs=[pl.BlockSpec((BM, C2), lambda i: (i, 0))]
        + [pl.BlockSpec((BM, QW), lambda i: (i, 0))] * 2
        + [
            pl.BlockSpec((BM, 1), lambda i: (i, 0)),
            pl.BlockSpec((C2, 8), lambda i: (0, 0)),
            pl.BlockSpec((1, C2), lambda i: (0, 0)),
            pl.BlockSpec((1, 8), lambda i: (0, 0)),
        ],
        out_specs=pl.BlockSpec((BM, 8), lambda i: (i, 0)),
        out_shape=jax.ShapeDtypeStruct((N, 8), jnp.float32),
    )(acc2, *hs2q, dis, Wro, b2, bro)


def _tcd_body(f_ref, wfc1_ref, bfc1_ref, g_ref, be_ref, wfc2_ref, bfc2_ref, o_ref):
    z = (
        jnp.dot(f_ref[...], wfc1_ref[...], preferred_element_type=jnp.float32)
        + bfc1_ref[...]
    )
    mu = jnp.mean(z, axis=0, keepdims=True)
    var = jnp.mean((z - mu) ** 2, axis=0, keepdims=True)
    zn = (z - mu) / jnp.sqrt(var + 1e-5) * g_ref[...] + be_ref[...]
    o_ref[...] = (
        jnp.dot(_mish(zn), wfc2_ref[...], preferred_element_type=jnp.float32)
        + bfc2_ref[...]
    )


def _tc_d(feats, Wfc1, bfc1, gamma, beta, Wfc2, bfc2):
    F = NUMROI * 8
    return pl.pallas_call(
        _tcd_body,
        in_specs=[
            pl.BlockSpec((B, F), lambda: (0, 0)),
            pl.BlockSpec((F, NUMROI), lambda: (0, 0)),
            pl.BlockSpec((1, NUMROI), lambda: (0, 0)),
            pl.BlockSpec((1, NUMROI), lambda: (0, 0)),
            pl.BlockSpec((1, NUMROI), lambda: (0, 0)),
            pl.BlockSpec((NUMROI, 2), lambda: (0, 0)),
            pl.BlockSpec((1, 2), lambda: (0, 0)),
        ],
        out_specs=pl.BlockSpec((B, 2), lambda: (0, 0)),
        out_shape=jax.ShapeDtypeStruct((B, 2), jnp.float32),
    )(feats, Wfc1, bfc1, gamma, beta, Wfc2, bfc2)


def kernel(x, edge_index, edge_attr, batch, W1, b1, W2, b2, Wro, bro,
           Wfc1, bfc1, gamma, beta, Wfc2, bfc2):
    row = edge_index[0].astype(jnp.int32)
    col = edge_index[1].astype(jnp.int32)
    w = edge_attr
    rowp = jnp.pad(row, (0, EPAD - E))
    colp = jnp.pad(col, (0, EPAD - E))
    wp = jnp.pad(w, (0, EPAD - E))

    deg = _sc_deg(col, w)
    *hs1q, dis = _tc_a(x, W1, deg[:N].reshape(-1, 1))
    acc1 = _sc_edge1(*hs1q, rowp, colp, wp)[:N]
    hs2q = _tc_b(acc1, hs1q, dis, W2, b1.reshape(1, -1))
    acc2 = _sc_edge2(*hs2q, rowp, colp, wp)[:N]
    r = _tc_c(acc2, hs2q, dis, Wro, b2.reshape(1, -1), bro.reshape(1, -1))
    feats = r.reshape(B, NUMROI * 8)
    return _tc_d(feats, Wfc1, bfc1.reshape(1, -1), gamma.reshape(1, -1),
                 beta.reshape(1, -1), Wfc2, bfc2.reshape(1, -1))


# speedup vs baseline: 10.1194x; 1.2946x over previous
import functools
import jax
import jax.numpy as jnp
from jax import lax
from jax.experimental import pallas as pl
from jax.experimental.pallas import tpu as pltpu
from jax.experimental.pallas import tpu_sc as plsc

N = 50000
E = 800000
NUMROI = 100
C1 = 128
C2 = 64
B = 500
BM = 1000

NPAD = 50176
EPS = E // 16
SUB = 2000


_NH = NPAD // 2
_RPT_D = _NH // 16


def _sc_deg_body(col_hbm, w_hbm, deg_hbm, colb, wb, zb, deg_sh):
    core = lax.axis_index("c")
    sub = lax.axis_index("s")
    lo = core * _NH

    def _z(i, _):
        zb[pl.ds(i * 16, 16)] = jnp.zeros((16,), jnp.float32)
        return 0

    lax.fori_loop(0, _RPT_D // 16, _z, 0)
    pltpu.sync_copy(zb, deg_sh.at[pl.ds(sub * _RPT_D, _RPT_D)])
    plsc.subcore_barrier()

    for c in range(EPS // SUB):
        base = sub * EPS + c * SUB
        pltpu.sync_copy(col_hbm.at[pl.ds(base, SUB)], colb)
        pltpu.sync_copy(w_hbm.at[pl.ds(base, SUB)], wb)

        def _mask(i, _):
            sl = pl.ds(i * 16, 16)
            cv = colb[sl]
            wv = wb[sl]
            m = (cv >= lo) & (cv < lo + _NH)
            colb[sl] = jnp.where(m, cv - lo, 0)
            wb[sl] = jnp.where(m, wv, 0.0)
            return 0

        lax.fori_loop(0, SUB // 16, _mask, 0)
        pltpu.sync_copy(wb, deg_sh.at[colb], add=True)

    plsc.subcore_barrier()
    pltpu.sync_copy(deg_sh.at[pl.ds(sub * _RPT_D, _RPT_D)], zb)
    pltpu.sync_copy(zb, deg_hbm.at[pl.ds(lo + sub * _RPT_D, _RPT_D)])


_sc_deg = functools.partial(
    pl.kernel,
    out_type=jax.ShapeDtypeStruct((NPAD,), jnp.float32),
    mesh=plsc.VectorSubcoreMesh(core_axis_name="c", subcore_axis_name="s"),
    scratch_types=[
        pltpu.VMEM((SUB,), jnp.int32),
        pltpu.VMEM((SUB,), jnp.float32),
        pltpu.VMEM((_RPT_D,), jnp.float32),
        pltpu.VMEM_SHARED((_NH,), jnp.float32),
    ],
)(_sc_deg_body)


QW = 32
EPAD = 819200
SUB2 = 2048
_EPT = EPAD // 16
_G = 128
_ZB = 112
_RPT = NPAD // 16


def _take16(v, idx):
    dn = lax.GatherDimensionNumbers(
        offset_dims=(), collapsed_slice_dims=(0,), start_index_map=(0,))
    return lax.gather(v, idx[:, None], dn, (1,),
                      mode=lax.GatherScatterMode.PROMISE_IN_BOUNDS)


def _edge_body(NQ, *refs):
    hs_q = refs[:NQ]
    row_hbm, col_hbm, w_hbm = refs[NQ:NQ + 3]
    acc_out = refs[NQ + 3]
    (rowb, colb, wb, colbuf, zbuf, flshb, rows_v, rows_w, acc_sh,
     sem, sem2) = refs[NQ + 4:]
    core = lax.axis_index("c")
    sub = lax.axis_index("s")

    for zi in range(_ZB):
        for cb in range(QW // 16):
            zbuf[zi, pl.ds(cb * 16, 16)] = jnp.zeros((16,), jnp.float32)

    for p in range(NQ // 2):
        for k in range(_RPT // _ZB):
            pltpu.sync_copy(zbuf, acc_sh.at[pl.ds(sub * _RPT + k * _ZB, _ZB), :])
        plsc.subcore_barrier()

        def _chunk(c, _):
            base = sub * _EPT + c * SUB2
            pltpu.sync_copy(row_hbm.at[pl.ds(base, SUB2)], rowb)
            pltpu.sync_copy(col_hbm.at[pl.ds(base, SUB2)], colb)
            pltpu.sync_copy(w_hbm.at[pl.ds(base, SUB2)], wb)

            def _start(g, buf, sm):
                gb = g * _G
                for q in (2 * p, 2 * p + 1):
                    @pl.when(core == (q % 2))
                    def _(q=q):
                        pltpu.async_copy(
                            hs_q[q].at[rowb.at[pl.ds(gb, _G)]], buf, sm)

            nloc = SUB2 // _G
            _start(0, rows_v, sem)
            for g in range(nloc):
                buf, sm = (rows_v, sem) if g % 2 == 0 else (rows_w, sem2)
                if g + 1 < nloc:
                    nbuf, nsm = (rows_v, sem) if (g + 1) % 2 == 0 else (rows_w, sem2)
                    _start(g + 1, nbuf, nsm)
                gb = g * _G
                pltpu.make_async_copy(
                    hs_q[2 * p].at[rowb.at[pl.ds(gb, _G)]], buf, sm).wait()
                for k in range(_G // 16):
                    colbuf[pl.ds(k * 16, 16)] = colb[pl.ds(gb + k * 16, 16)]

                def _s(jv, _, gb=gb, buf=buf):
                    wv = wb[pl.ds(gb + jv * 16, 16)]
                    for l in range(16):
                        ws = _take16(wv, jnp.full((16,), l, jnp.int32))
                        j = jv * 16 + l
                        for cb in range(QW // 16):
                            sl = pl.ds(cb * 16, 16)
                            buf[j, sl] = buf[j, sl] * ws
                    return 0

                lax.fori_loop(0, _G // 16, _s, 0)
                pltpu.sync_copy(buf, acc_sh.at[colbuf], add=True)
            return 0

        lax.fori_loop(0, _EPT // SUB2, _chunk, 0)
        plsc.subcore_barrier()
        for q in (2 * p, 2 * p + 1):
            @pl.when(core == (q % 2))
            def _(q=q):
                for k in range(_RPT // _ZB):
                    r0 = sub * _RPT + k * _ZB
                    pltpu.sync_copy(acc_sh.at[pl.ds(r0, _ZB), :], flshb)
                    pltpu.sync_copy(
                        flshb,
                        acc_out.at[pl.ds(r0, _ZB), pl.ds(q * QW, QW)])
        if p + 1 < NQ // 2:
            plsc.subcore_barrier()


def _sc_edge(NQ):
    return functools.partial(
        pl.kernel,
        out_type=jax.ShapeDtypeStruct((NPAD, NQ * QW), jnp.float32),
        mesh=plsc.VectorSubcoreMesh(core_axis_name="c", subcore_axis_name="s"),
        compiler_params=pltpu.CompilerParams(use_tc_tiling_on_sc=False),
        scratch_types=[
            pltpu.VMEM((SUB2,), jnp.int32),
            pltpu.VMEM((SUB2,), jnp.int32),
            pltpu.VMEM((SUB2,), jnp.float32),
            pltpu.VMEM((_G,), jnp.int32),
            pltpu.VMEM((_ZB, QW), jnp.float32),
            pltpu.VMEM((_ZB, QW), jnp.float32),
            pltpu.VMEM((_G, QW), jnp.float32),
            pltpu.VMEM((_G, QW), jnp.float32),
            pltpu.VMEM_SHARED((NPAD, QW), jnp.float32),
            pltpu.SemaphoreType.DMA,
            pltpu.SemaphoreType.DMA,
        ],
    )(functools.partial(_edge_body, NQ))


_sc_edge1 = _sc_edge(4)
_sc_edge2 = _sc_edge(2)


def _mish(x):
    return x * jnp.tanh(jax.nn.softplus(x))


def _tca_body(x_ref, w1_ref, deg_ref, h0, h1, h2, h3, dis_ref):
    dis = lax.rsqrt(deg_ref[...] + 1.0)
    h = jnp.dot(x_ref[...], w1_ref[...], preferred_element_type=jnp.float32)
    for q, href in enumerate((h0, h1, h2, h3)):
        href[...] = h[:, q * QW:(q + 1) * QW] * dis
    dis_ref[...] = dis


def _tc_a(x, W1, deg):
    grid = N // BM
    return pl.pallas_call(
        _tca_body,
        grid=(grid,),
        in_specs=[
            pl.BlockSpec((BM, NUMROI), lambda i: (i, 0)),
            pl.BlockSpec((NUMROI, C1), lambda i: (0, 0)),
            pl.BlockSpec((BM, 1), lambda i: (i, 0)),
        ],
        out_specs=[pl.BlockSpec((BM, QW), lambda i: (i, 0))] * 4
        + [pl.BlockSpec((BM, 1), lambda i: (i, 0))],
        out_shape=[jax.ShapeDtypeStruct((N, QW), jnp.float32)] * 4
        + [jax.ShapeDtypeStruct((N, 1), jnp.float32)],
    )(x, W1, deg)


def _tcb_body(a_ref, h0, h1, h2, h3, dis_ref, w2_ref, b1_ref, o0, o1):
    dis = dis_ref[...]
    hs = jnp.concatenate([h[...] for h in (h0, h1, h2, h3)], axis=1)
    z = dis * (a_ref[...] + hs) + b1_ref[...]
    g = _mish(z)
    hh = jnp.dot(g, w2_ref[...], preferred_element_type=jnp.float32) * dis
    o0[...] = hh[:, :QW]
    o1[...] = hh[:, QW:]


def _tc_b(acc1, hs1q, dis, W2, b1):
    grid = N // BM
    return pl.pallas_call(
        _tcb_body,
        grid=(grid,),
        in_specs=[pl.BlockSpec((BM, C1), lambda i: (i, 0))]
        + [pl.BlockSpec((BM, QW), lambda i: (i, 0))] * 4
        + [
            pl.BlockSpec((BM, 1), lambda i: (i, 0)),
            pl.BlockSpec((C1, C2), lambda i: (0, 0)),
            pl.BlockSpec((1, C1), lambda i: (0, 0)),
        ],
        out_specs=[pl.BlockSpec((BM, QW), lambda i: (i, 0))] * 2,
        out_shape=[jax.ShapeDtypeStruct((N, QW), jnp.float32)] * 2,
    )(acc1, *hs1q, dis, W2, b1)


def _tcc_body(a_ref, h0, h1, dis_ref, wro_ref, b2_ref, bro_ref, r_ref):
    dis = dis_ref[...]
    hs = jnp.concatenate([h[...] for h in (h0, h1)], axis=1)
    z = dis * (a_ref[...] + hs) + b2_ref[...]
    g = _mish(z)
    r_ref[...] = _mish(
        jnp.dot(g, wro_ref[...], preferred_element_type=jnp.float32) + bro_ref[...]
    )


def _tc_c(acc2, hs2q, dis, Wro, b2, bro):
    grid = N // BM
    return pl.pallas_call(
        _tcc_body,
        grid=(grid,),
        in_specs=[pl.BlockSpec((BM, C2), lambda i: (i, 0))]
        + [pl.BlockSpec((BM, QW), lambda i: (i, 0))] * 2
        + [
            pl.BlockSpec((BM, 1), lambda i: (i, 0)),
            pl.BlockSpec((C2, 8), lambda i: (0, 0)),
            pl.BlockSpec((1, C2), lambda i: (0, 0)),
            pl.BlockSpec((1, 8), lambda i: (0, 0)),
        ],
        out_specs=pl.BlockSpec((BM, 8), lambda i: (i, 0)),
        out_shape=jax.ShapeDtypeStruct((N, 8), jnp.float32),
    )(acc2, *hs2q, dis, Wro, b2, bro)


def _tcd_body(f_ref, wfc1_ref, bfc1_ref, g_ref, be_ref, wfc2_ref, bfc2_ref, o_ref):
    z = (
        jnp.dot(f_ref[...], wfc1_ref[...], preferred_element_type=jnp.float32)
        + bfc1_ref[...]
    )
    mu = jnp.mean(z, axis=0, keepdims=True)
    var = jnp.mean((z - mu) ** 2, axis=0, keepdims=True)
    zn = (z - mu) / jnp.sqrt(var + 1e-5) * g_ref[...] + be_ref[...]
    o_ref[...] = (
        jnp.dot(_mish(zn), wfc2_ref[...], preferred_element_type=jnp.float32)
        + bfc2_ref[...]
    )


def _tc_d(feats, Wfc1, bfc1, gamma, beta, Wfc2, bfc2):
    F = NUMROI * 8
    return pl.pallas_call(
        _tcd_body,
        in_specs=[
            pl.BlockSpec((B, F), lambda: (0, 0)),
            pl.BlockSpec((F, NUMROI), lambda: (0, 0)),
            pl.BlockSpec((1, NUMROI), lambda: (0, 0)),
            pl.BlockSpec((1, NUMROI), lambda: (0, 0)),
            pl.BlockSpec((1, NUMROI), lambda: (0, 0)),
            pl.BlockSpec((NUMROI, 2), lambda: (0, 0)),
            pl.BlockSpec((1, 2), lambda: (0, 0)),
        ],
        out_specs=pl.BlockSpec((B, 2), lambda: (0, 0)),
        out_shape=jax.ShapeDtypeStruct((B, 2), jnp.float32),
    )(feats, Wfc1, bfc1, gamma, beta, Wfc2, bfc2)


def kernel(x, edge_index, edge_attr, batch, W1, b1, W2, b2, Wro, bro,
           Wfc1, bfc1, gamma, beta, Wfc2, bfc2):
    row = edge_index[0].astype(jnp.int32)
    col = edge_index[1].astype(jnp.int32)
    w = edge_attr
    rowp = jnp.pad(row, (0, EPAD - E))
    colp = jnp.pad(col, (0, EPAD - E))
    wp = jnp.pad(w, (0, EPAD - E))

    deg = _sc_deg(col, w)
    *hs1q, dis = _tc_a(x, W1, deg[:N].reshape(-1, 1))
    acc1 = _sc_edge1(*hs1q, rowp, colp, wp)[:N]
    hs2q = _tc_b(acc1, hs1q, dis, W2, b1.reshape(1, -1))
    acc2 = _sc_edge2(*hs2q, rowp, colp, wp)[:N]
    r = _tc_c(acc2, hs2q, dis, Wro, b2.reshape(1, -1), bro.reshape(1, -1))
    feats = r.reshape(B, NUMROI * 8)
    return _tc_d(feats, Wfc1, bfc1.reshape(1, -1), gamma.reshape(1, -1),
                 beta.reshape(1, -1), Wfc2, bfc2.reshape(1, -1))


# speedup vs baseline: 10.3807x; 1.0258x over previous
import functools
import jax
import jax.numpy as jnp
from jax import lax
from jax.experimental import pallas as pl
from jax.experimental.pallas import tpu as pltpu
from jax.experimental.pallas import tpu_sc as plsc

N = 50000
E = 800000
NUMROI = 100
C1 = 128
C2 = 64
B = 500
BM = 1000

NPAD = 50176
EPS = E // 16
SUB = 2000


_NH = NPAD // 2
_RPT_D = _NH // 16


def _sc_deg_body(col_hbm, w_hbm, deg_hbm, colb, wb, zb, deg_sh):
    core = lax.axis_index("c")
    sub = lax.axis_index("s")
    lo = core * _NH

    def _z(i, _):
        zb[pl.ds(i * 16, 16)] = jnp.zeros((16,), jnp.float32)
        return 0

    lax.fori_loop(0, _RPT_D // 16, _z, 0)
    pltpu.sync_copy(zb, deg_sh.at[pl.ds(sub * _RPT_D, _RPT_D)])
    plsc.subcore_barrier()

    for c in range(EPS // SUB):
        base = sub * EPS + c * SUB
        pltpu.sync_copy(col_hbm.at[pl.ds(base, SUB)], colb)
        pltpu.sync_copy(w_hbm.at[pl.ds(base, SUB)], wb)

        def _mask(i, _):
            sl = pl.ds(i * 16, 16)
            cv = colb[sl]
            wv = wb[sl]
            m = (cv >= lo) & (cv < lo + _NH)
            colb[sl] = jnp.where(m, cv - lo, 0)
            wb[sl] = jnp.where(m, wv, 0.0)
            return 0

        lax.fori_loop(0, SUB // 16, _mask, 0)
        pltpu.sync_copy(wb, deg_sh.at[colb], add=True)

    plsc.subcore_barrier()
    pltpu.sync_copy(deg_sh.at[pl.ds(sub * _RPT_D, _RPT_D)], zb)
    pltpu.sync_copy(zb, deg_hbm.at[pl.ds(lo + sub * _RPT_D, _RPT_D)])


_sc_deg = functools.partial(
    pl.kernel,
    out_type=jax.ShapeDtypeStruct((NPAD,), jnp.float32),
    mesh=plsc.VectorSubcoreMesh(core_axis_name="c", subcore_axis_name="s"),
    scratch_types=[
        pltpu.VMEM((SUB,), jnp.int32),
        pltpu.VMEM((SUB,), jnp.float32),
        pltpu.VMEM((_RPT_D,), jnp.float32),
        pltpu.VMEM_SHARED((_NH,), jnp.float32),
    ],
)(_sc_deg_body)


QW = 32
EPAD = 819200
SUB2 = 2048
_EPT = EPAD // 16
_G = 128
_ZB = 112
_RPT = NPAD // 16


def _take16(v, idx):
    dn = lax.GatherDimensionNumbers(
        offset_dims=(), collapsed_slice_dims=(0,), start_index_map=(0,))
    return lax.gather(v, idx[:, None], dn, (1,),
                      mode=lax.GatherScatterMode.PROMISE_IN_BOUNDS)


def _edge_body(NQ, *refs):
    hs_q = refs[:NQ]
    row_hbm, col_hbm, w_hbm = refs[NQ:NQ + 3]
    acc_out = refs[NQ + 3]
    (rowb, colb, wb, colbuf, zbuf, flshb, rows_v, rows_w, acc_sh,
     sem, sem2) = refs[NQ + 4:]
    core = lax.axis_index("c")
    sub = lax.axis_index("s")

    for zi in range(_ZB):
        for cb in range(QW // 16):
            zbuf[zi, pl.ds(cb * 16, 16)] = jnp.zeros((16,), jnp.float32)

    for p in range(NQ // 2):
        for k in range(_RPT // _ZB):
            pltpu.sync_copy(zbuf, acc_sh.at[pl.ds(sub * _RPT + k * _ZB, _ZB), :])
        plsc.subcore_barrier()

        def _chunk(c, _):
            base = sub * _EPT + c * SUB2
            pltpu.sync_copy(row_hbm.at[pl.ds(base, SUB2)], rowb)
            pltpu.sync_copy(col_hbm.at[pl.ds(base, SUB2)], colb)
            pltpu.sync_copy(w_hbm.at[pl.ds(base, SUB2)], wb)

            def _start(g, buf, sm):
                gb = g * _G
                for q in (2 * p, 2 * p + 1):
                    @pl.when(core == (q % 2))
                    def _(q=q):
                        pltpu.async_copy(
                            hs_q[q].at[rowb.at[pl.ds(gb, _G)]], buf, sm)

            nloc = SUB2 // _G
            _start(0, rows_v, sem)
            for g in range(nloc):
                buf, sm = (rows_v, sem) if g % 2 == 0 else (rows_w, sem2)
                if g + 1 < nloc:
                    nbuf, nsm = (rows_v, sem) if (g + 1) % 2 == 0 else (rows_w, sem2)
                    _start(g + 1, nbuf, nsm)
                gb = g * _G
                pltpu.make_async_copy(
                    hs_q[2 * p].at[rowb.at[pl.ds(gb, _G)]], buf, sm).wait()
                for k in range(_G // 16):
                    colbuf[pl.ds(k * 16, 16)] = colb[pl.ds(gb + k * 16, 16)]

                def _s(jv, _, gb=gb, buf=buf):
                    wv = wb[pl.ds(gb + jv * 16, 16)]
                    for l in range(16):
                        ws = _take16(wv, jnp.full((16,), l, jnp.int32))
                        j = jv * 16 + l
                        for cb in range(QW // 16):
                            sl = pl.ds(cb * 16, 16)
                            buf[j, sl] = buf[j, sl] * ws
                    return 0

                lax.fori_loop(0, _G // 16, _s, 0)
                pltpu.sync_copy(buf, acc_sh.at[colbuf], add=True)
            return 0

        lax.fori_loop(0, _EPT // SUB2, _chunk, 0)
        plsc.subcore_barrier()
        for q in (2 * p, 2 * p + 1):
            @pl.when(core == (q % 2))
            def _(q=q):
                for k in range(_RPT // _ZB):
                    r0 = sub * _RPT + k * _ZB
                    pltpu.sync_copy(acc_sh.at[pl.ds(r0, _ZB), :], flshb)
                    pltpu.sync_copy(
                        flshb,
                        acc_out.at[pl.ds(r0, _ZB), pl.ds(q * QW, QW)])
        if p + 1 < NQ // 2:
            plsc.subcore_barrier()


def _sc_edge(NQ):
    return functools.partial(
        pl.kernel,
        out_type=jax.ShapeDtypeStruct((NPAD, NQ * QW), jnp.float32),
        mesh=plsc.VectorSubcoreMesh(core_axis_name="c", subcore_axis_name="s"),
        compiler_params=pltpu.CompilerParams(use_tc_tiling_on_sc=False),
        scratch_types=[
            pltpu.VMEM((SUB2,), jnp.int32),
            pltpu.VMEM((SUB2,), jnp.int32),
            pltpu.VMEM((SUB2,), jnp.float32),
            pltpu.VMEM((_G,), jnp.int32),
            pltpu.VMEM((_ZB, QW), jnp.float32),
            pltpu.VMEM((_ZB, QW), jnp.float32),
            pltpu.VMEM((_G, QW), jnp.float32),
            pltpu.VMEM((_G, QW), jnp.float32),
            pltpu.VMEM_SHARED((NPAD, QW), jnp.float32),
            pltpu.SemaphoreType.DMA,
            pltpu.SemaphoreType.DMA,
        ],
    )(functools.partial(_edge_body, NQ))


_sc_edge1 = _sc_edge(4)
_sc_edge2 = _sc_edge(2)


def _mish(x):
    return x * jnp.tanh(jax.nn.softplus(x))


def _tca_body(x_ref, w1_ref, deg_ref, h0, h1, h2, h3, dis_ref):
    dis = lax.rsqrt(deg_ref[...] + 1.0)
    h = jnp.dot(x_ref[...], w1_ref[...], preferred_element_type=jnp.float32)
    for q, href in enumerate((h0, h1, h2, h3)):
        href[...] = h[:, q * QW:(q + 1) * QW] * dis
    dis_ref[...] = dis


def _tc_a(x, W1, deg):
    grid = N // BM
    return pl.pallas_call(
        _tca_body,
        grid=(grid,),
        in_specs=[
            pl.BlockSpec((BM, NUMROI), lambda i: (i, 0)),
            pl.BlockSpec((NUMROI, C1), lambda i: (0, 0)),
            pl.BlockSpec((BM, 1), lambda i: (i, 0)),
        ],
        out_specs=[pl.BlockSpec((BM, QW), lambda i: (i, 0))] * 4
        + [pl.BlockSpec((BM, 1), lambda i: (i, 0))],
        out_shape=[jax.ShapeDtypeStruct((N, QW), jnp.float32)] * 4
        + [jax.ShapeDtypeStruct((N, 1), jnp.float32)],
    )(x, W1, deg)


def _tcb_body(a_ref, h0, h1, h2, h3, dis_ref, w2_ref, b1_ref, o0, o1):
    dis = dis_ref[...]
    hs = jnp.concatenate([h[...] for h in (h0, h1, h2, h3)], axis=1)
    z = dis * (a_ref[...] + hs) + b1_ref[...]
    g = _mish(z)
    hh = jnp.dot(g, w2_ref[...], preferred_element_type=jnp.float32) * dis
    o0[...] = hh[:, :QW]
    o1[...] = hh[:, QW:]


def _tc_b(acc1, hs1q, dis, W2, b1):
    grid = N // BM
    return pl.pallas_call(
        _tcb_body,
        grid=(grid,),
        in_specs=[pl.BlockSpec((BM, C1), lambda i: (i, 0))]
        + [pl.BlockSpec((BM, QW), lambda i: (i, 0))] * 4
        + [
            pl.BlockSpec((BM, 1), lambda i: (i, 0)),
            pl.BlockSpec((C1, C2), lambda i: (0, 0)),
            pl.BlockSpec((1, C1), lambda i: (0, 0)),
        ],
        out_specs=[pl.BlockSpec((BM, QW), lambda i: (i, 0))] * 2,
        out_shape=[jax.ShapeDtypeStruct((N, QW), jnp.float32)] * 2,
    )(acc1, *hs1q, dis, W2, b1)


def _tcc_body(a_ref, h0, h1, dis_ref, wro_ref, b2_ref, bro_ref, r_ref):
    dis = dis_ref[...]
    hs = jnp.concatenate([h[...] for h in (h0, h1)], axis=1)
    z = dis * (a_ref[...] + hs) + b2_ref[...]
    g = _mish(z)
    r_ref[...] = _mish(
        jnp.dot(g, wro_ref[...], preferred_element_type=jnp.float32) + bro_ref[...]
    )


def _tc_c(acc2, hs2q, dis, Wro, b2, bro):
    grid = N // BM
    return pl.pallas_call(
        _tcc_body,
        grid=(grid,),
        in_specs=[pl.BlockSpec((BM, C2), lambda i: (i, 0))]
        + [pl.BlockSpec((BM, QW), lambda i: (i, 0))] * 2
        + [
            pl.BlockSpec((BM, 1), lambda i: (i, 0)),
            pl.BlockSpec((C2, 8), lambda i: (0, 0)),
            pl.BlockSpec((1, C2), lambda i: (0, 0)),
            pl.BlockSpec((1, 8), lambda i: (0, 0)),
        ],
        out_specs=pl.BlockSpec((BM, 8), lambda i: (i, 0)),
        out_shape=jax.ShapeDtypeStruct((N, 8), jnp.float32),
    )(acc2, *hs2q, dis, Wro, b2, bro)


def _tcd_body(f_ref, wfc1_ref, bfc1_ref, g_ref, be_ref, wfc2_ref, bfc2_ref, o_ref):
    z = (
        jnp.dot(f_ref[...], wfc1_ref[...], preferred_element_type=jnp.float32)
        + bfc1_ref[...]
    )
    mu = jnp.mean(z, axis=0, keepdims=True)
    var = jnp.mean((z - mu) ** 2, axis=0, keepdims=True)
    zn = (z - mu) / jnp.sqrt(var + 1e-5) * g_ref[...] + be_ref[...]
    o_ref[...] = (
        jnp.dot(_mish(zn), wfc2_ref[...], preferred_element_type=jnp.float32)
        + bfc2_ref[...]
    )


def _tc_d(feats, Wfc1, bfc1, gamma, beta, Wfc2, bfc2):
    F = NUMROI * 8
    return pl.pallas_call(
        _tcd_body,
        in_specs=[
            pl.BlockSpec((B, F), lambda: (0, 0)),
            pl.BlockSpec((F, NUMROI), lambda: (0, 0)),
            pl.BlockSpec((1, NUMROI), lambda: (0, 0)),
            pl.BlockSpec((1, NUMROI), lambda: (0, 0)),
            pl.BlockSpec((1, NUMROI), lambda: (0, 0)),
            pl.BlockSpec((NUMROI, 2), lambda: (0, 0)),
            pl.BlockSpec((1, 2), lambda: (0, 0)),
        ],
        out_specs=pl.BlockSpec((B, 2), lambda: (0, 0)),
        out_shape=jax.ShapeDtypeStruct((B, 2), jnp.float32),
    )(feats, Wfc1, bfc1, gamma, beta, Wfc2, bfc2)


def kernel(x, edge_index, edge_attr, batch, W1, b1, W2, b2, Wro, bro,
           Wfc1, bfc1, gamma, beta, Wfc2, bfc2):
    row = edge_index[0].astype(jnp.int32)
    col = edge_index[1].astype(jnp.int32)
    w = edge_attr
    rowp = jnp.pad(row, (0, EPAD - E))
    colp = jnp.pad(col, (0, EPAD - E))
    wp = jnp.pad(w, (0, EPAD - E))

    deg = _sc_deg(col, w)
    *hs1q, dis = _tc_a(x, W1, deg.reshape(-1, 1))
    acc1 = _sc_edge1(*hs1q, rowp, colp, wp)
    hs2q = _tc_b(acc1, hs1q, dis, W2, b1.reshape(1, -1))
    acc2 = _sc_edge2(*hs2q, rowp, colp, wp)
    r = _tc_c(acc2, hs2q, dis, Wro, b2.reshape(1, -1), bro.reshape(1, -1))
    feats = r.reshape(B, NUMROI * 8)
    return _tc_d(feats, Wfc1, bfc1.reshape(1, -1), gamma.reshape(1, -1),
                 beta.reshape(1, -1), Wfc2, bfc2.reshape(1, -1))


# speedup vs baseline: 10.6302x; 1.0240x over previous
import functools
import jax
import jax.numpy as jnp
from jax import lax
from jax.experimental import pallas as pl
from jax.experimental.pallas import tpu as pltpu
from jax.experimental.pallas import tpu_sc as plsc

N = 50000
E = 800000
NUMROI = 100
C1 = 128
C2 = 64
B = 500
BM = 1000

NPAD = 50176
EPS = E // 16
SUB = 2000


_NH = NPAD // 2
_RPT_D = _NH // 16


def _sc_deg_body(col_hbm, w_hbm, deg_hbm, colb, wb, zb, deg_sh):
    core = lax.axis_index("c")
    sub = lax.axis_index("s")
    lo = core * _NH

    def _z(i, _):
        zb[pl.ds(i * 16, 16)] = jnp.zeros((16,), jnp.float32)
        return 0

    lax.fori_loop(0, _RPT_D // 16, _z, 0)
    pltpu.sync_copy(zb, deg_sh.at[pl.ds(sub * _RPT_D, _RPT_D)])
    plsc.subcore_barrier()

    for c in range(EPS // SUB):
        base = sub * EPS + c * SUB
        pltpu.sync_copy(col_hbm.at[pl.ds(base, SUB)], colb)
        pltpu.sync_copy(w_hbm.at[pl.ds(base, SUB)], wb)

        def _mask(i, _):
            sl = pl.ds(i * 16, 16)
            cv = colb[sl]
            wv = wb[sl]
            m = (cv >= lo) & (cv < lo + _NH)
            colb[sl] = jnp.where(m, cv - lo, 0)
            wb[sl] = jnp.where(m, wv, 0.0)
            return 0

        lax.fori_loop(0, SUB // 16, _mask, 0)
        pltpu.sync_copy(wb, deg_sh.at[colb], add=True)

    plsc.subcore_barrier()
    pltpu.sync_copy(deg_sh.at[pl.ds(sub * _RPT_D, _RPT_D)], zb)
    pltpu.sync_copy(zb, deg_hbm.at[pl.ds(lo + sub * _RPT_D, _RPT_D)])


_sc_deg = functools.partial(
    pl.kernel,
    out_type=jax.ShapeDtypeStruct((NPAD,), jnp.float32),
    mesh=plsc.VectorSubcoreMesh(core_axis_name="c", subcore_axis_name="s"),
    scratch_types=[
        pltpu.VMEM((SUB,), jnp.int32),
        pltpu.VMEM((SUB,), jnp.float32),
        pltpu.VMEM((_RPT_D,), jnp.float32),
        pltpu.VMEM_SHARED((_NH,), jnp.float32),
    ],
)(_sc_deg_body)


QW = 32
EPAD = 819200
SUB2 = 3200
_EPT = EPAD // 16
_G = 128
_ZB = 112
_RPT = NPAD // 16


def _take16(v, idx):
    dn = lax.GatherDimensionNumbers(
        offset_dims=(), collapsed_slice_dims=(0,), start_index_map=(0,))
    return lax.gather(v, idx[:, None], dn, (1,),
                      mode=lax.GatherScatterMode.PROMISE_IN_BOUNDS)


def _edge_body(NQ, *refs):
    hs_q = refs[:NQ]
    row_hbm, col_hbm, w_hbm = refs[NQ:NQ + 3]
    acc_out = refs[NQ + 3]
    (rowb, colb, wb, colbuf, zbuf, flshb, rows_v, rows_w, acc_sh,
     sem, sem2) = refs[NQ + 4:]
    core = lax.axis_index("c")
    sub = lax.axis_index("s")

    for zi in range(_ZB):
        for cb in range(QW // 16):
            zbuf[zi, pl.ds(cb * 16, 16)] = jnp.zeros((16,), jnp.float32)

    for p in range(NQ // 2):
        for k in range(_RPT // _ZB):
            pltpu.sync_copy(zbuf, acc_sh.at[pl.ds(sub * _RPT + k * _ZB, _ZB), :])
        plsc.subcore_barrier()

        def _chunk(c, _):
            base = sub * _EPT + c * SUB2
            pltpu.sync_copy(row_hbm.at[pl.ds(base, SUB2)], rowb)
            pltpu.sync_copy(col_hbm.at[pl.ds(base, SUB2)], colb)
            pltpu.sync_copy(w_hbm.at[pl.ds(base, SUB2)], wb)

            def _start(g, buf, sm):
                gb = g * _G
                for q in (2 * p, 2 * p + 1):
                    @pl.when(core == (q % 2))
                    def _(q=q):
                        pltpu.async_copy(
                            hs_q[q].at[rowb.at[pl.ds(gb, _G)]], buf, sm)

            nloc = SUB2 // _G
            _start(0, rows_v, sem)
            for g in range(nloc):
                buf, sm = (rows_v, sem) if g % 2 == 0 else (rows_w, sem2)
                if g + 1 < nloc:
                    nbuf, nsm = (rows_v, sem) if (g + 1) % 2 == 0 else (rows_w, sem2)
                    _start(g + 1, nbuf, nsm)
                gb = g * _G
                pltpu.make_async_copy(
                    hs_q[2 * p].at[rowb.at[pl.ds(gb, _G)]], buf, sm).wait()
                for k in range(_G // 16):
                    colbuf[pl.ds(k * 16, 16)] = colb[pl.ds(gb + k * 16, 16)]

                def _s(jv, _, gb=gb, buf=buf):
                    wv = wb[pl.ds(gb + jv * 16, 16)]
                    for l in range(16):
                        ws = _take16(wv, jnp.full((16,), l, jnp.int32))
                        j = jv * 16 + l
                        for cb in range(QW // 16):
                            sl = pl.ds(cb * 16, 16)
                            buf[j, sl] = buf[j, sl] * ws
                    return 0

                lax.fori_loop(0, _G // 16, _s, 0)
                pltpu.sync_copy(buf, acc_sh.at[colbuf], add=True)
            return 0

        lax.fori_loop(0, _EPT // SUB2, _chunk, 0)
        plsc.subcore_barrier()
        for q in (2 * p, 2 * p + 1):
            @pl.when(core == (q % 2))
            def _(q=q):
                for k in range(_RPT // _ZB):
                    r0 = sub * _RPT + k * _ZB
                    pltpu.sync_copy(acc_sh.at[pl.ds(r0, _ZB), :], flshb)
                    pltpu.sync_copy(
                        flshb,
                        acc_out.at[pl.ds(r0, _ZB), pl.ds(q * QW, QW)])
        if p + 1 < NQ // 2:
            plsc.subcore_barrier()


def _sc_edge(NQ):
    return functools.partial(
        pl.kernel,
        out_type=jax.ShapeDtypeStruct((NPAD, NQ * QW), jnp.float32),
        mesh=plsc.VectorSubcoreMesh(core_axis_name="c", subcore_axis_name="s"),
        compiler_params=pltpu.CompilerParams(use_tc_tiling_on_sc=False),
        scratch_types=[
            pltpu.VMEM((SUB2,), jnp.int32),
            pltpu.VMEM((SUB2,), jnp.int32),
            pltpu.VMEM((SUB2,), jnp.float32),
            pltpu.VMEM((_G,), jnp.int32),
            pltpu.VMEM((_ZB, QW), jnp.float32),
            pltpu.VMEM((_ZB, QW), jnp.float32),
            pltpu.VMEM((_G, QW), jnp.float32),
            pltpu.VMEM((_G, QW), jnp.float32),
            pltpu.VMEM_SHARED((NPAD, QW), jnp.float32),
            pltpu.SemaphoreType.DMA,
            pltpu.SemaphoreType.DMA,
        ],
    )(functools.partial(_edge_body, NQ))


_sc_edge1 = _sc_edge(4)
_sc_edge2 = _sc_edge(2)


def _mish(x):
    return x * jnp.tanh(jax.nn.softplus(x))


def _tca_body(x_ref, w1_ref, deg_ref, h0, h1, h2, h3, dis_ref):
    dis = lax.rsqrt(deg_ref[...] + 1.0)
    h = jnp.dot(x_ref[...], w1_ref[...], preferred_element_type=jnp.float32)
    for q, href in enumerate((h0, h1, h2, h3)):
        href[...] = h[:, q * QW:(q + 1) * QW] * dis
    dis_ref[...] = dis


def _tc_a(x, W1, deg):
    grid = N // BM
    return pl.pallas_call(
        _tca_body,
        grid=(grid,),
        in_specs=[
            pl.BlockSpec((BM, NUMROI), lambda i: (i, 0)),
            pl.BlockSpec((NUMROI, C1), lambda i: (0, 0)),
            pl.BlockSpec((BM, 1), lambda i: (i, 0)),
        ],
        out_specs=[pl.BlockSpec((BM, QW), lambda i: (i, 0))] * 4
        + [pl.BlockSpec((BM, 1), lambda i: (i, 0))],
        out_shape=[jax.ShapeDtypeStruct((N, QW), jnp.float32)] * 4
        + [jax.ShapeDtypeStruct((N, 1), jnp.float32)],
    )(x, W1, deg)


def _tcb_body(a_ref, h0, h1, h2, h3, dis_ref, w2_ref, b1_ref, o0, o1):
    dis = dis_ref[...]
    hs = jnp.concatenate([h[...] for h in (h0, h1, h2, h3)], axis=1)
    z = dis * (a_ref[...] + hs) + b1_ref[...]
    g = _mish(z)
    hh = jnp.dot(g, w2_ref[...], preferred_element_type=jnp.float32) * dis
    o0[...] = hh[:, :QW]
    o1[...] = hh[:, QW:]


def _tc_b(acc1, hs1q, dis, W2, b1):
    grid = N // BM
    return pl.pallas_call(
        _tcb_body,
        grid=(grid,),
        in_specs=[pl.BlockSpec((BM, C1), lambda i: (i, 0))]
        + [pl.BlockSpec((BM, QW), lambda i: (i, 0))] * 4
        + [
            pl.BlockSpec((BM, 1), lambda i: (i, 0)),
            pl.BlockSpec((C1, C2), lambda i: (0, 0)),
            pl.BlockSpec((1, C1), lambda i: (0, 0)),
        ],
        out_specs=[pl.BlockSpec((BM, QW), lambda i: (i, 0))] * 2,
        out_shape=[jax.ShapeDtypeStruct((N, QW), jnp.float32)] * 2,
    )(acc1, *hs1q, dis, W2, b1)


def _tcc_body(a_ref, h0, h1, dis_ref, wro_ref, b2_ref, bro_ref, r_ref):
    dis = dis_ref[...]
    hs = jnp.concatenate([h[...] for h in (h0, h1)], axis=1)
    z = dis * (a_ref[...] + hs) + b2_ref[...]
    g = _mish(z)
    r_ref[...] = _mish(
        jnp.dot(g, wro_ref[...], preferred_element_type=jnp.float32) + bro_ref[...]
    )


def _tc_c(acc2, hs2q, dis, Wro, b2, bro):
    grid = N // BM
    return pl.pallas_call(
        _tcc_body,
        grid=(grid,),
        in_specs=[pl.BlockSpec((BM, C2), lambda i: (i, 0))]
        + [pl.BlockSpec((BM, QW), lambda i: (i, 0))] * 2
        + [
            pl.BlockSpec((BM, 1), lambda i: (i, 0)),
            pl.BlockSpec((C2, 8), lambda i: (0, 0)),
            pl.BlockSpec((1, C2), lambda i: (0, 0)),
            pl.BlockSpec((1, 8), lambda i: (0, 0)),
        ],
        out_specs=pl.BlockSpec((BM, 8), lambda i: (i, 0)),
        out_shape=jax.ShapeDtypeStruct((N, 8), jnp.float32),
    )(acc2, *hs2q, dis, Wro, b2, bro)


def _tcd_body(f_ref, wfc1_ref, bfc1_ref, g_ref, be_ref, wfc2_ref, bfc2_ref, o_ref):
    z = (
        jnp.dot(f_ref[...], wfc1_ref[...], preferred_element_type=jnp.float32)
        + bfc1_ref[...]
    )
    mu = jnp.mean(z, axis=0, keepdims=True)
    var = jnp.mean((z - mu) ** 2, axis=0, keepdims=True)
    zn = (z - mu) / jnp.sqrt(var + 1e-5) * g_ref[...] + be_ref[...]
    o_ref[...] = (
        jnp.dot(_mish(zn), wfc2_ref[...], preferred_element_type=jnp.float32)
        + bfc2_ref[...]
    )


def _tc_d(feats, Wfc1, bfc1, gamma, beta, Wfc2, bfc2):
    F = NUMROI * 8
    return pl.pallas_call(
        _tcd_body,
        in_specs=[
            pl.BlockSpec((B, F), lambda: (0, 0)),
            pl.BlockSpec((F, NUMROI), lambda: (0, 0)),
            pl.BlockSpec((1, NUMROI), lambda: (0, 0)),
            pl.BlockSpec((1, NUMROI), lambda: (0, 0)),
            pl.BlockSpec((1, NUMROI), lambda: (0, 0)),
            pl.BlockSpec((NUMROI, 2), lambda: (0, 0)),
            pl.BlockSpec((1, 2), lambda: (0, 0)),
        ],
        out_specs=pl.BlockSpec((B, 2), lambda: (0, 0)),
        out_shape=jax.ShapeDtypeStruct((B, 2), jnp.float32),
    )(feats, Wfc1, bfc1, gamma, beta, Wfc2, bfc2)


def kernel(x, edge_index, edge_attr, batch, W1, b1, W2, b2, Wro, bro,
           Wfc1, bfc1, gamma, beta, Wfc2, bfc2):
    row = edge_index[0].astype(jnp.int32)
    col = edge_index[1].astype(jnp.int32)
    w = edge_attr
    rowp = jnp.pad(row, (0, EPAD - E))
    colp = jnp.pad(col, (0, EPAD - E))
    wp = jnp.pad(w, (0, EPAD - E))

    deg = _sc_deg(col, w)
    *hs1q, dis = _tc_a(x, W1, deg.reshape(-1, 1))
    acc1 = _sc_edge1(*hs1q, rowp, colp, wp)
    hs2q = _tc_b(acc1, hs1q, dis, W2, b1.reshape(1, -1))
    acc2 = _sc_edge2(*hs2q, rowp, colp, wp)
    r = _tc_c(acc2, hs2q, dis, Wro, b2.reshape(1, -1), bro.reshape(1, -1))
    feats = r.reshape(B, NUMROI * 8)
    return _tc_d(feats, Wfc1, bfc1.reshape(1, -1), gamma.reshape(1, -1),
                 beta.reshape(1, -1), Wfc2, bfc2.reshape(1, -1))
